# Initial kernel scaffold; baseline (speedup 1.0000x reference)
import jax
import jax.numpy as jnp
from jax.experimental import pallas as pl


def kernel(responses):
    raise NotImplementedError("write your pallas kernel here")


# speedup vs baseline: 2.7034x; 2.7034x over previous
import jax
import jax.numpy as jnp
from jax import lax
from jax.experimental import pallas as pl
from jax.experimental.pallas import tpu as pltpu

SLIP = 0.1
GUESS = 0.3
TRAIN_P = 0.1
LEARN_P = 0.5

LANES = 128
ROWS = 16
B_BLK = ROWS * LANES
TCHUNK = 128
RENORM = 16


def _bkt_body(resp_ref, pred_ref, true_ref, cs_ref, ps_ref):
    T = resp_ref.shape[1]
    Tm1 = T - 1
    n_chunks = T // TCHUNK

    true_ref[...] = resp_ref[:, 1:T].astype(jnp.float32)

    u = jnp.full((ROWS, LANES), LEARN_P, jnp.float32)
    v = jnp.full((ROWS, LANES), 1.0 - LEARN_P, jnp.float32)

    for tc in range(n_chunks):
        col0 = tc * TCHUNK
        for r in range(ROWS):
            tile = resp_ref[r * LANES:(r + 1) * LANES,
                            col0:col0 + TCHUNK]
            cs_ref[:, r, :] = jnp.transpose((tile == 1).astype(jnp.float32))

        def macro_step(i, carry):
            u, v = carry
            base = i * RENORM
            for j in range(RENORM):
                c = cs_ref[base + j]
                a = 0.1 + 0.8 * c
                pb = 0.07 - 0.04 * c
                qb = 0.63 - 0.36 * c
                u1 = a * u + pb * v
                v1 = qb * v
                den = u1 + v1
                r0 = pl.reciprocal(den, approx=True)
                r1 = r0 * (2.0 - den * r0)
                k = u1 * r1
                ps_ref[base + j] = k
                if j == RENORM - 1:
                    u, v = k, v1 * r1
                else:
                    u, v = u1, v1
            return (u, v)

        u, v = lax.fori_loop(0, TCHUNK // RENORM, macro_step, (u, v))

        w = min(TCHUNK, Tm1 - col0)
        for r in range(ROWS):
            tp = jnp.transpose(ps_ref[:, r, :])
            pred_ref[r * LANES:(r + 1) * LANES, col0:col0 + w] = tp[:, :w]


def kernel(responses):
    responses = responses.astype(jnp.int32)
    B, T = responses.shape
    Tm1 = T - 1

    grid = (B // B_BLK,)
    out_shape = (
        jax.ShapeDtypeStruct((B, Tm1), jnp.float32),
        jax.ShapeDtypeStruct((B, Tm1), jnp.float32),
    )
    pred, true = pl.pallas_call(
        _bkt_body,
        out_shape=out_shape,
        grid=grid,
        in_specs=[pl.BlockSpec((B_BLK, T), lambda i: (i, 0))],
        out_specs=[
            pl.BlockSpec((B_BLK, Tm1), lambda i: (i, 0)),
            pl.BlockSpec((B_BLK, Tm1), lambda i: (i, 0)),
        ],
        scratch_shapes=[
            pltpu.VMEM((TCHUNK, ROWS, LANES), jnp.float32),
            pltpu.VMEM((TCHUNK, ROWS, LANES), jnp.float32),
        ],
        compiler_params=pltpu.CompilerParams(
            dimension_semantics=("parallel",),
        ),
    )(responses)
    return {"pred": pred, "true": true}


# speedup vs baseline: 2.7936x; 1.0334x over previous
import jax
import jax.numpy as jnp
from jax import lax
from jax.experimental import pallas as pl
from jax.experimental.pallas import tpu as pltpu

SLIP = 0.1
GUESS = 0.3
TRAIN_P = 0.1
LEARN_P = 0.5

LANES = 128
ROWS = 16
B_BLK = ROWS * LANES
TCHUNK = 128
RENORM = 16


def _bkt_body(resp_ref, pred_ref, true_ref, cs_ref, us_ref, vs_ref):
    T = resp_ref.shape[1]
    Tm1 = T - 1
    n_chunks = T // TCHUNK

    true_ref[...] = resp_ref[:, 1:T].astype(jnp.float32)

    u = jnp.full((ROWS, LANES), LEARN_P, jnp.float32)
    v = jnp.full((ROWS, LANES), 1.0 - LEARN_P, jnp.float32)

    for tc in range(n_chunks):
        col0 = tc * TCHUNK
        for r in range(ROWS):
            tile = resp_ref[r * LANES:(r + 1) * LANES,
                            col0:col0 + TCHUNK]
            cs_ref[:, r, :] = jnp.transpose((tile == 1).astype(jnp.float32))

        def macro_step(i, carry):
            u, v = carry
            base = i * RENORM
            for j in range(RENORM):
                c = cs_ref[base + j]
                a = 0.1 + 0.8 * c
                pb = 0.075 - 0.05 * a
                qb = 9.0 * pb
                u1 = a * u + pb * v
                v1 = qb * v
                us_ref[base + j] = u1
                vs_ref[base + j] = v1
                if j == RENORM - 1:
                    s = pl.reciprocal(u1 + v1, approx=True)
                    u, v = u1 * s, v1 * s
                else:
                    u, v = u1, v1
            return (u, v)

        u, v = lax.fori_loop(0, TCHUNK // RENORM, macro_step, (u, v))

        uall = us_ref[...]
        kall = uall * pl.reciprocal(uall + vs_ref[...], approx=True)
        us_ref[...] = kall

        w = min(TCHUNK, Tm1 - col0)
        for r in range(ROWS):
            tp = jnp.transpose(us_ref[:, r, :])
            pred_ref[r * LANES:(r + 1) * LANES, col0:col0 + w] = tp[:, :w]


def kernel(responses):
    responses = responses.astype(jnp.int32)
    B, T = responses.shape
    Tm1 = T - 1

    grid = (B // B_BLK,)
    out_shape = (
        jax.ShapeDtypeStruct((B, Tm1), jnp.float32),
        jax.ShapeDtypeStruct((B, Tm1), jnp.float32),
    )
    pred, true = pl.pallas_call(
        _bkt_body,
        out_shape=out_shape,
        grid=grid,
        in_specs=[pl.BlockSpec((B_BLK, T), lambda i: (i, 0))],
        out_specs=[
            pl.BlockSpec((B_BLK, Tm1), lambda i: (i, 0)),
            pl.BlockSpec((B_BLK, Tm1), lambda i: (i, 0)),
        ],
        scratch_shapes=[
            pltpu.VMEM((TCHUNK, ROWS, LANES), jnp.float32),
            pltpu.VMEM((TCHUNK, ROWS, LANES), jnp.float32),
            pltpu.VMEM((TCHUNK, ROWS, LANES), jnp.float32),
        ],
        compiler_params=pltpu.CompilerParams(
            dimension_semantics=("parallel",),
        ),
    )(responses)
    return {"pred": pred, "true": true}
